# baseline (speedup 1.0000x reference)
import jax
import jax.numpy as jnp
from jax.experimental import pallas as pl
from jax.experimental.pallas import tpu as pltpu

_BIG = 400
_TAIL = (200, 104, 56, 40)
_NBUF = 2
_NOUT = 2


def _block_schedule(n):
    big_rows = n - sum(_TAIL)
    blocks = [(off, _BIG) for off in range(0, big_rows, _BIG)]
    off = big_rows
    for size in _TAIL:
        blocks.append((off, size))
        off += size
    return blocks


def _gcn_manual(seq_ref, w_ref, adj_hbm, out_hbm,
                support_ref, abuf, obuf, asem, osem):
    blocks = _block_schedule(adj_hbm.shape[0])
    nblocks = len(blocks)

    def adj_copy(b):
        off, size = blocks[b]
        return pltpu.make_async_copy(
            adj_hbm.at[pl.ds(off, size), :],
            abuf.at[b % _NBUF, pl.ds(0, size), :],
            asem.at[b % _NBUF],
        )

    def out_copy(b):
        off, size = blocks[b]
        return pltpu.make_async_copy(
            obuf.at[b % _NOUT, pl.ds(0, size), :],
            out_hbm.at[pl.ds(off, size), :],
            osem.at[b % _NOUT],
        )

    for b in range(_NBUF):
        adj_copy(b).start()

    support_ref[...] = jnp.dot(
        seq_ref[...], w_ref[...], preferred_element_type=jnp.float32
    )

    for b in range(nblocks):
        size = blocks[b][1]
        adj_copy(b).wait()
        if b >= _NOUT:
            out_copy(b - _NOUT).wait()
        obuf[b % _NOUT, pl.ds(0, size), :] = jnp.tanh(
            jnp.dot(abuf[b % _NBUF, pl.ds(0, size), :], support_ref[...],
                    preferred_element_type=jnp.float32)
        )
        out_copy(b).start()
        if b + _NBUF < nblocks:
            adj_copy(b + _NBUF).start()

    for b in range(nblocks - _NOUT, nblocks):
        out_copy(b).wait()


def kernel(seq, adj, weight):
    n, in_ft = seq.shape
    out_ft = weight.shape[1]
    return pl.pallas_call(
        _gcn_manual,
        in_specs=[
            pl.BlockSpec((n, in_ft), lambda: (0, 0)),
            pl.BlockSpec((in_ft, out_ft), lambda: (0, 0)),
            pl.BlockSpec(memory_space=pl.ANY),
        ],
        out_specs=pl.BlockSpec(memory_space=pl.ANY),
        out_shape=jax.ShapeDtypeStruct((n, out_ft), jnp.float32),
        scratch_shapes=[
            pltpu.VMEM((n, out_ft), jnp.float32),
            pltpu.VMEM((_NBUF, _BIG, n), jnp.float32),
            pltpu.VMEM((_NOUT, _BIG, out_ft), jnp.float32),
            pltpu.SemaphoreType.DMA((_NBUF,)),
            pltpu.SemaphoreType.DMA((_NOUT,)),
        ],
    )(seq, weight, adj)


# speedup vs baseline: 1.0460x; 1.0460x over previous
import jax
import jax.numpy as jnp
from jax.experimental import pallas as pl
from jax.experimental.pallas import tpu as pltpu

_BIG = 320
_TAIL = (200, 104, 56, 40)
_NBUF = 3
_NOUT = 2


def _block_schedule(n):
    big_rows = n - sum(_TAIL)
    blocks = [(off, _BIG) for off in range(0, big_rows, _BIG)]
    off = big_rows
    for size in _TAIL:
        blocks.append((off, size))
        off += size
    return blocks


def _gcn_manual(seq_ref, w_ref, adj_hbm, out_hbm,
                support_ref, abuf, obuf, asem, osem):
    blocks = _block_schedule(adj_hbm.shape[0])
    nblocks = len(blocks)

    def adj_copy(b):
        off, size = blocks[b]
        return pltpu.make_async_copy(
            adj_hbm.at[pl.ds(off, size), :],
            abuf.at[b % _NBUF, pl.ds(0, size), :],
            asem.at[b % _NBUF],
        )

    def out_copy(b):
        off, size = blocks[b]
        return pltpu.make_async_copy(
            obuf.at[b % _NOUT, pl.ds(0, size), :],
            out_hbm.at[pl.ds(off, size), :],
            osem.at[b % _NOUT],
        )

    for b in range(_NBUF):
        adj_copy(b).start()

    support_ref[...] = jnp.dot(
        seq_ref[...], w_ref[...], preferred_element_type=jnp.float32
    )

    for b in range(nblocks):
        size = blocks[b][1]
        adj_copy(b).wait()
        if b >= _NOUT:
            out_copy(b - _NOUT).wait()
        obuf[b % _NOUT, pl.ds(0, size), :] = jnp.tanh(
            jnp.dot(abuf[b % _NBUF, pl.ds(0, size), :], support_ref[...],
                    preferred_element_type=jnp.float32)
        )
        out_copy(b).start()
        if b + _NBUF < nblocks:
            adj_copy(b + _NBUF).start()

    for b in range(nblocks - _NOUT, nblocks):
        out_copy(b).wait()


def kernel(seq, adj, weight):
    n, in_ft = seq.shape
    out_ft = weight.shape[1]
    return pl.pallas_call(
        _gcn_manual,
        in_specs=[
            pl.BlockSpec((n, in_ft), lambda: (0, 0)),
            pl.BlockSpec((in_ft, out_ft), lambda: (0, 0)),
            pl.BlockSpec(memory_space=pl.ANY),
        ],
        out_specs=pl.BlockSpec(memory_space=pl.ANY),
        out_shape=jax.ShapeDtypeStruct((n, out_ft), jnp.float32),
        scratch_shapes=[
            pltpu.VMEM((n, out_ft), jnp.float32),
            pltpu.VMEM((_NBUF, _BIG, n), jnp.float32),
            pltpu.VMEM((_NOUT, _BIG, out_ft), jnp.float32),
            pltpu.SemaphoreType.DMA((_NBUF,)),
            pltpu.SemaphoreType.DMA((_NOUT,)),
        ],
    )(seq, weight, adj)


# speedup vs baseline: 1.0844x; 1.0366x over previous
import jax
import jax.numpy as jnp
from jax.experimental import pallas as pl
from jax.experimental.pallas import tpu as pltpu

_BI = 400


def _gcn_block(seq_ref, w_ref, adj_ref, out_ref, support_ref):
    @pl.when(pl.program_id(0) == 0)
    def _():
        support_ref[...] = jnp.dot(
            seq_ref[...], w_ref[...], preferred_element_type=jnp.float32
        )

    out_ref[...] = jnp.tanh(
        jnp.dot(adj_ref[...], support_ref[...], preferred_element_type=jnp.float32)
    )


def kernel(seq, adj, weight):
    n, in_ft = seq.shape
    out_ft = weight.shape[1]
    return pl.pallas_call(
        _gcn_block,
        grid=(n // _BI,),
        in_specs=[
            pl.BlockSpec((n, in_ft), lambda i: (0, 0)),
            pl.BlockSpec((in_ft, out_ft), lambda i: (0, 0)),
            pl.BlockSpec((_BI, n), lambda i: (i, 0)),
        ],
        out_specs=pl.BlockSpec((_BI, out_ft), lambda i: (i, 0)),
        out_shape=jax.ShapeDtypeStruct((n, out_ft), jnp.float32),
        scratch_shapes=[pltpu.VMEM((n, out_ft), jnp.float32)],
    )(seq, weight, adj)
